# baseline (device time: 278109 ns/iter reference)
import jax
import jax.numpy as jnp
from jax import lax
from jax.experimental import pallas as pl
from jax.experimental.pallas import tpu as pltpu

N_X = 2
N_Y = 2
C = 16


def kernel(x):
    m, n = x.shape
    n_out = n // N_X
    half = m // N_Y
    rows = half // C

    def body(x_ref, out_ref, vin_a, vin_b, vcast_a, vcast_b,
             ina_sems, inb_sems, lca_sems, lcb_sems,
             dsend, drecv, fsend, frecv):
        mx = lax.axis_index("x")
        my = lax.axis_index("y")
        ox = 1 - mx
        oy = 1 - my

        barrier = pltpu.get_barrier_semaphore()
        for nbr in [(ox, my), (mx, oy)]:
            pl.semaphore_signal(barrier, inc=1, device_id=nbr,
                                device_id_type=pl.DeviceIdType.MESH)
        pl.semaphore_wait(barrier, 2)

        base_a = my * half
        base_b = oy * half

        def stage_in(k):
            ia = pltpu.make_async_copy(
                x_ref.at[pl.ds(base_a + k * rows, rows), :],
                vin_a.at[k % 2], ina_sems.at[k % 2])
            ib = pltpu.make_async_copy(
                x_ref.at[pl.ds(base_b + k * rows, rows), :],
                vin_b.at[k % 2], inb_sems.at[k % 2])
            ia.start()
            ib.start()
            return ia, ib

        ins = [stage_in(0), stage_in(1)]
        lcs = []
        rdmas = []
        fwds = []

        for k in range(C):
            ins[k][0].wait()
            ins[k][1].wait()
            if k >= 2:
                lcs[k - 2][0].wait()
                lcs[k - 2][1].wait()
                rdmas[k - 2].wait_send()

            vcast_a[k % 2] = vin_a[k % 2].astype(jnp.bfloat16)
            vcast_b[k % 2] = vin_b[k % 2].astype(jnp.bfloat16)

            lca = pltpu.make_async_copy(
                vcast_a.at[k % 2, :, pl.ds(mx * n_out, n_out)],
                out_ref.at[pl.ds(mx * m + base_a + k * rows, rows), :],
                lca_sems.at[k % 2])
            lcb = pltpu.make_async_copy(
                vcast_b.at[k % 2, :, pl.ds(mx * n_out, n_out)],
                out_ref.at[pl.ds(mx * m + base_b + k * rows, rows), :],
                lcb_sems.at[k % 2])
            lca.start()
            lcb.start()
            lcs.append((lca, lcb))

            d = pltpu.make_async_remote_copy(
                src_ref=vcast_a.at[k % 2, :, pl.ds(ox * n_out, n_out)],
                dst_ref=out_ref.at[pl.ds(mx * m + base_a + k * rows, rows), :],
                send_sem=dsend.at[k], recv_sem=drecv.at[k],
                device_id=(ox, my), device_id_type=pl.DeviceIdType.MESH,
            )
            d.start()
            rdmas.append(d)

            if k + 2 < C:
                ins.append(stage_in(k + 2))

            d.wait_recv()
            rcv_r = ox * m + base_a + k * rows
            f = pltpu.make_async_remote_copy(
                src_ref=out_ref.at[pl.ds(rcv_r, rows), :],
                dst_ref=out_ref.at[pl.ds(rcv_r, rows), :],
                send_sem=fsend.at[k], recv_sem=frecv.at[k],
                device_id=(mx, oy), device_id_type=pl.DeviceIdType.MESH,
            )
            f.start()
            fwds.append(f)

        for f in fwds:
            f.wait_recv()
        for k in (C - 2, C - 1):
            lcs[k][0].wait()
            lcs[k][1].wait()
            rdmas[k].wait_send()
        for f in fwds:
            f.wait_send()

    return pl.pallas_call(
        body,
        out_shape=jax.ShapeDtypeStruct((N_X * m, n_out), jnp.bfloat16),
        in_specs=[pl.BlockSpec(memory_space=pl.ANY)],
        out_specs=pl.BlockSpec(memory_space=pl.ANY),
        scratch_shapes=[
            pltpu.VMEM((2, rows, n), jnp.float32),
            pltpu.VMEM((2, rows, n), jnp.float32),
            pltpu.VMEM((2, rows, n), jnp.bfloat16),
            pltpu.VMEM((2, rows, n), jnp.bfloat16),
            pltpu.SemaphoreType.DMA((2,)),
            pltpu.SemaphoreType.DMA((2,)),
            pltpu.SemaphoreType.DMA((2,)),
            pltpu.SemaphoreType.DMA((2,)),
            pltpu.SemaphoreType.DMA((C,)),
            pltpu.SemaphoreType.DMA((C,)),
            pltpu.SemaphoreType.DMA((C,)),
            pltpu.SemaphoreType.DMA((C,)),
        ],
        compiler_params=pltpu.CompilerParams(
            collective_id=0, vmem_limit_bytes=96 * 1024 * 1024
        ),
    )(x)


# device time: 237451 ns/iter; 1.1712x vs baseline; 1.1712x over previous
import jax
import jax.numpy as jnp
from jax import lax
from jax.experimental import pallas as pl
from jax.experimental.pallas import tpu as pltpu

N_X = 2
N_Y = 2
C = 32
D = 1
S = 3


def kernel(x):
    m, n = x.shape
    n_out = n // N_X
    half = m // N_Y
    rows = half // C

    def body(x_ref, out_ref, vin_a, vin_b, vcast_a, vcast_b, landing,
             ina_sems, inb_sems, lca_sems, lcb_sems, ld_sems,
             dsend, drecv, fsend, frecv):
        mx = lax.axis_index("x")
        my = lax.axis_index("y")
        ox = 1 - mx
        oy = 1 - my

        base_a = my * half
        base_b = oy * half

        def stage_in(k):
            ia = pltpu.make_async_copy(
                x_ref.at[pl.ds(base_a + k * rows, rows), :],
                vin_a.at[k % 2], ina_sems.at[k % 2])
            ib = pltpu.make_async_copy(
                x_ref.at[pl.ds(base_b + k * rows, rows), :],
                vin_b.at[k % 2], inb_sems.at[k % 2])
            ia.start()
            ib.start()
            return ia, ib

        ins = [stage_in(0), stage_in(1)]

        barrier = pltpu.get_barrier_semaphore()
        for nbr in [(ox, my), (mx, oy)]:
            pl.semaphore_signal(barrier, inc=1, device_id=nbr,
                                device_id_type=pl.DeviceIdType.MESH)
        pl.semaphore_wait(barrier, 2)

        lcs = []
        rdmas = []
        relays = []

        def relay(k):
            rdmas[k].wait_recv()
            lo = pltpu.make_async_copy(
                landing.at[k],
                out_ref.at[pl.ds(ox * m + base_a + k * rows, rows), :],
                ld_sems.at[k])
            lo.start()
            f = pltpu.make_async_remote_copy(
                src_ref=landing.at[k],
                dst_ref=out_ref.at[pl.ds(ox * m + base_a + k * rows, rows), :],
                send_sem=fsend.at[k], recv_sem=frecv.at[k],
                device_id=(mx, oy), device_id_type=pl.DeviceIdType.MESH,
            )
            f.start()
            relays.append((lo, f))

        for k in range(C):
            ins[k][0].wait()
            ins[k][1].wait()
            if k >= S:
                lcs[k - S][0].wait()
                lcs[k - S][1].wait()
                rdmas[k - S].wait_send()

            vcast_a[k % S] = vin_a[k % 2].astype(jnp.bfloat16)
            vcast_b[k % S] = vin_b[k % 2].astype(jnp.bfloat16)

            lca = pltpu.make_async_copy(
                vcast_a.at[k % S, :, pl.ds(mx * n_out, n_out)],
                out_ref.at[pl.ds(mx * m + base_a + k * rows, rows), :],
                lca_sems.at[k % S])
            lcb = pltpu.make_async_copy(
                vcast_b.at[k % S, :, pl.ds(mx * n_out, n_out)],
                out_ref.at[pl.ds(mx * m + base_b + k * rows, rows), :],
                lcb_sems.at[k % S])
            lca.start()
            lcb.start()
            lcs.append((lca, lcb))

            d = pltpu.make_async_remote_copy(
                src_ref=vcast_a.at[k % S, :, pl.ds(ox * n_out, n_out)],
                dst_ref=landing.at[k],
                send_sem=dsend.at[k], recv_sem=drecv.at[k],
                device_id=(ox, my), device_id_type=pl.DeviceIdType.MESH,
            )
            d.start()
            rdmas.append(d)

            if k + 2 < C:
                ins.append(stage_in(k + 2))

            if k >= D:
                relay(k - D)

        for k in range(C - D, C):
            relay(k)
        for lo, f in relays:
            f.wait_recv()
        for k in range(C - S, C):
            rdmas[k].wait_send()
            lcs[k][0].wait()
            lcs[k][1].wait()
        for lo, f in relays:
            lo.wait()
            f.wait_send()

    return pl.pallas_call(
        body,
        out_shape=jax.ShapeDtypeStruct((N_X * m, n_out), jnp.bfloat16),
        in_specs=[pl.BlockSpec(memory_space=pl.ANY)],
        out_specs=pl.BlockSpec(memory_space=pl.ANY),
        scratch_shapes=[
            pltpu.VMEM((2, rows, n), jnp.float32),
            pltpu.VMEM((2, rows, n), jnp.float32),
            pltpu.VMEM((S, rows, n), jnp.bfloat16),
            pltpu.VMEM((S, rows, n), jnp.bfloat16),
            pltpu.VMEM((C, rows, n_out), jnp.bfloat16),
            pltpu.SemaphoreType.DMA((2,)),
            pltpu.SemaphoreType.DMA((2,)),
            pltpu.SemaphoreType.DMA((S,)),
            pltpu.SemaphoreType.DMA((S,)),
            pltpu.SemaphoreType.DMA((C,)),
            pltpu.SemaphoreType.DMA((C,)),
            pltpu.SemaphoreType.DMA((C,)),
            pltpu.SemaphoreType.DMA((C,)),
            pltpu.SemaphoreType.DMA((C,)),
        ],
        compiler_params=pltpu.CompilerParams(
            collective_id=0, vmem_limit_bytes=100 * 1024 * 1024
        ),
    )(x)
